# baseline (speedup 1.0000x reference)
import jax
import jax.numpy as jnp
from jax.experimental import pallas as pl

GROUP_NUMS = 512
GROUP_SIZE = 128


def _knn_group(points, centers, k):
    d2 = (jnp.sum(centers ** 2, axis=-1, keepdims=True)
          + jnp.sum(points ** 2, axis=-1)[:, None, :]
          - 2.0 * jnp.einsum('bpd,bnd->bpn', centers, points))
    _, idx = jax.lax.top_k(-d2, k)
    grouped = jax.vmap(lambda pts, i: pts[i])(points, idx)
    return grouped, idx


def kernel(pointclouds):
    B, N, dim = pointclouds.shape
    cidx = jnp.linspace(0.0, N - 1, GROUP_NUMS).astype(jnp.int32)
    centers = pointclouds[:, cidx, :]
    grouped, _ = _knn_group(pointclouds, centers, GROUP_SIZE)
    means = jnp.mean(grouped, axis=2)
    grouped2, _ = _knn_group(pointclouds, means, GROUP_SIZE)
    centered = grouped2 - jnp.mean(grouped2, axis=2, keepdims=True)
    cov = jnp.einsum('bpki,bpkj->bpij', centered, centered) / float(GROUP_SIZE)
    curvatures, coord_frames = jnp.linalg.eigh(cov)
    main_axis = coord_frames[:, :, :, 2][:, :, None, :]
    main_axis_expanded = jnp.broadcast_to(main_axis, grouped2.shape)
    enhanced = jnp.concatenate([grouped2, main_axis_expanded], axis=-1)
    enhanced = enhanced.reshape(B, -1, enhanced.shape[-1])
    idx = jnp.linspace(0.0, enhanced.shape[1] - 1, N).astype(jnp.int32)
    sampled = enhanced[:, idx, :]
    return sampled


# speedup vs baseline: 3.9805x; 3.9805x over previous
import functools

import jax
import jax.numpy as jnp
import numpy as np
from jax import lax
from jax.experimental import pallas as pl
from jax.experimental.pallas import tpu as pltpu
from jax.experimental.pallas import tpu_sc as plsc

P = 512
K = 128
N = 16384
B = 8
R = 32
S = 32

_INF_BITS = 0x7F800000


def _sortable_key(d2):
    bu = lax.bitcast_convert_type(d2, jnp.uint32)
    sign = jnp.uint32(0x80000000)
    return jnp.where(bu >= sign, ~bu, bu | sign)


def _key_to_float(key):
    sign = jnp.uint32(0x80000000)
    bu = jnp.where(key >= sign, key ^ sign, ~key)
    return lax.bitcast_convert_type(bu, jnp.float32)


def _search_topk_threshold(bits, lanes, need_total):
    rshape = (bits.shape[0], 1)

    def vbody(_, lohi):
        lo, hi = lohi
        mid = lo + lax.div(hi - lo, jnp.uint32(2))
        cnt = jnp.sum(jnp.where(bits <= mid, 1, 0), axis=1, keepdims=True)
        ge = cnt >= need_total
        return jnp.where(ge, lo, mid + 1), jnp.where(ge, mid, hi)

    lo0 = jnp.zeros(rshape, jnp.uint32)
    hi0 = jnp.full(rshape, 0xFFFFFFFF, jnp.uint32)
    lo, _ = lax.fori_loop(0, 32, vbody, (lo0, hi0))
    v = lo

    cnt_lt = jnp.sum(jnp.where(bits < v, 1, 0), axis=1, keepdims=True)
    need = need_total - cnt_lt
    eq = bits == v

    def ibody(_, lohi):
        lo, hi = lohi
        mid = lo + lax.div(hi - lo, 2)
        cnt = jnp.sum(jnp.where(eq & (lanes <= mid), 1, 0), axis=1,
                      keepdims=True)
        ge = cnt >= need
        return jnp.where(ge, lo, mid + 1), jnp.where(ge, mid, hi)

    lo0i = jnp.zeros(rshape, jnp.int32)
    hi0i = jnp.full(rshape, N - 1, jnp.int32)
    lo2, _ = lax.fori_loop(0, 15, ibody, (lo0i, hi0i))
    return v, lo2


def _tc_body(p3_ref, c3_ref, d2_ref, thr_ref, cut_ref):
    pmat = p3_ref[0]
    px = pmat[0:1, :]
    py = pmat[1:2, :]
    pz = pmat[2:3, :]
    pp = (px * px + py * py) + pz * pz

    lanes = lax.broadcasted_iota(jnp.int32, (R, N), 1)

    def raw_d2(cmat, cc):
        cp = jnp.dot(cmat, pmat)
        return cc + pp - 2.0 * cp

    cmat1 = c3_ref[0]
    cx = cmat1[:, 0:1]
    cy = cmat1[:, 1:2]
    cz = cmat1[:, 2:3]
    cc1 = (cx * cx + cy * cy) + cz * cz
    d2c1 = raw_d2(cmat1, cc1)
    bits1 = _sortable_key(d2c1)
    v1, c1 = _search_topk_threshold(bits1, lanes, K)
    mask1 = (bits1 < v1) | ((bits1 == v1) & (lanes <= c1))

    mx = jnp.sum(jnp.where(mask1, px, 0.0), axis=1, keepdims=True) / float(K)
    my = jnp.sum(jnp.where(mask1, py, 0.0), axis=1, keepdims=True) / float(K)
    mz = jnp.sum(jnp.where(mask1, pz, 0.0), axis=1, keepdims=True) / float(K)

    cmat2 = jnp.concatenate([mx, my, mz], axis=1)
    cc2 = (mx * mx + my * my) + mz * mz
    d2c2 = raw_d2(cmat2, cc2)
    bits2 = _sortable_key(d2c2)
    v2, c2 = _search_topk_threshold(bits2, lanes, K)

    d2_ref[0] = d2c2
    thr_ref[0, 0] = _key_to_float(v2)
    cut_ref[0, 0] = c2


def _tc_pass(p3, c3):
    grid = (B, P // R)
    return pl.pallas_call(
        _tc_body,
        grid=grid,
        in_specs=[
            pl.BlockSpec((1, 3, N), lambda b, pc: (b, 0, 0)),
            pl.BlockSpec((1, R, 3), lambda b, pc: (b, pc, 0)),
        ],
        out_specs=[
            pl.BlockSpec((1, R, N), lambda b, pc: (b, pc, 0)),
            pl.BlockSpec((1, 1, R, 1), lambda b, pc: (b, pc, 0, 0)),
            pl.BlockSpec((1, 1, R, 1), lambda b, pc: (b, pc, 0, 0)),
        ],
        out_shape=[
            jax.ShapeDtypeStruct((B, P, N), jnp.float32),
            jax.ShapeDtypeStruct((B, P // R, R, 1), jnp.float32),
            jax.ShapeDtypeStruct((B, P // R, R, 1), jnp.int32),
        ],
    )(p3, c3)


_ROWS_PER_TILE = (B * P) // 32


def _cmpex(ka, va, kb, vb):
    le = ka <= kb
    kl = jnp.where(le, ka, kb)
    kh = jnp.where(le, kb, ka)
    vl = jnp.where(le, va, vb)
    vh = jnp.where(le, vb, va)
    return kl, vl, kh, vh


def _bitonic_merge(ks, vs):
    m = len(ks)
    ks = list(ks)
    vs = list(vs)
    d = m // 2
    while d >= 1:
        for i in range(m):
            if (i % (2 * d)) < d:
                kl, vl, kh, vh = _cmpex(ks[i], vs[i], ks[i + d], vs[i + d])
                ks[i], vs[i] = kl, vl
                ks[i + d], vs[i + d] = kh, vh
        d //= 2
    for i in range(m):
        ks[i], vs[i] = plsc.sort_key_val(ks[i], vs[i])
    return ks, vs


def _merge_sorted(ka, va, kb, vb):
    kb2 = [lax.rev(x, (0,)) for x in reversed(kb)]
    vb2 = [lax.rev(x, (0,)) for x in reversed(vb)]
    return _bitonic_merge(ka + kb2, va + vb2)


def _sort128(ks, vs):
    ks = list(ks)
    vs = list(vs)
    for i in range(8):
        ks[i], vs[i] = plsc.sort_key_val(ks[i], vs[i])
    width = 1
    while width < 8:
        nk, nv = [], []
        for i in range(0, 8, 2 * width):
            mk, mv = _merge_sorted(ks[i:i + width], vs[i:i + width],
                                   ks[i + width:i + 2 * width],
                                   vs[i + width:i + 2 * width])
            nk += mk
            nv += mv
        ks, vs = nk, nv
        width *= 2
    return ks, vs


def _sc_body(d2_hbm, thr_hbm, cut_hbm, px_hbm, py_hbm, pz_hbm, js_hbm,
             out3_hbm, g3_hbm,
             px_v, py_v, pz_v, js_v, thr_v, cut_v, d2_v, cand_v, cand_i,
             sidx_v, out3_v, g3_v):
    c = lax.axis_index("c")
    s = lax.axis_index("s")
    wid = s * 2 + c
    row0 = wid * _ROWS_PER_TILE
    b = lax.div(row0, P)
    pbase = lax.rem(row0, P)

    pltpu.sync_copy(px_hbm.at[b], px_v)
    pltpu.sync_copy(py_hbm.at[b], py_v)
    pltpu.sync_copy(pz_hbm.at[b], pz_v)
    pltpu.sync_copy(js_hbm.at[pl.ds(pbase, _ROWS_PER_TILE)], js_v)
    pltpu.sync_copy(thr_hbm.at[pl.ds(row0, _ROWS_PER_TILE)],
                    thr_v.at[pl.ds(0, _ROWS_PER_TILE)])
    pltpu.sync_copy(cut_hbm.at[pl.ds(row0, _ROWS_PER_TILE)],
                    cut_v.at[pl.ds(0, _ROWS_PER_TILE)])

    iota = lax.iota(jnp.int32, 16)

    def row_body(r, carry):
        pltpu.sync_copy(d2_hbm.at[row0 + r], d2_v)
        thr = jnp.full((16,), thr_v[pl.ds(r, 16)][0], jnp.float32)
        cut = jnp.full((16,), cut_v[pl.ds(r, 16)][0], jnp.int32)

        def chunk(i, off):
            v = d2_v[pl.ds(i * 16, 16)]
            gi = i * 16 + iota
            m = (v < thr) | ((v == thr) & (gi <= cut))
            csum = plsc.cumsum(jnp.where(m, 1, 0))
            pos = off + csum - 1
            plsc.store_scatter(cand_v, [pos], v, mask=m)
            plsc.store_scatter(cand_i, [pos], gi, mask=m)
            return off + csum[15]

        lax.fori_loop(0, N // 16, chunk, jnp.int32(0))

        ks = [cand_v[pl.ds(j * 16, 16)] for j in range(8)]
        vs = [cand_i[pl.ds(j * 16, 16)] for j in range(8)]
        ks, vs = _sort128(ks, vs)
        for j in range(8):
            sidx_v[pl.ds(j * 16, 16)] = vs[j]

        xs = [plsc.load_gather(px_v, [vs[j]]) for j in range(8)]
        ys = [plsc.load_gather(py_v, [vs[j]]) for j in range(8)]
        zs = [plsc.load_gather(pz_v, [vs[j]]) for j in range(8)]

        for j in range(8):
            gpos = (j * 16 + iota) * 3
            plsc.store_scatter(g3_v, [gpos], xs[j])
            plsc.store_scatter(g3_v, [gpos + 1], ys[j])
            plsc.store_scatter(g3_v, [gpos + 2], zs[j])
        pltpu.sync_copy(g3_v, g3_hbm.at[pl.ds((row0 + r) * (K * 3), K * 3)])

        base = r * (S * 3)
        for h in range(2):
            jv = js_v[r, pl.ds(h * 16, 16)]
            nv = plsc.load_gather(sidx_v, [jv])
            gx = plsc.load_gather(px_v, [nv])
            gy = plsc.load_gather(py_v, [nv])
            gz = plsc.load_gather(pz_v, [nv])
            pos = base + (h * 16 + iota) * 3
            plsc.store_scatter(out3_v, [pos], gx)
            plsc.store_scatter(out3_v, [pos + 1], gy)
            plsc.store_scatter(out3_v, [pos + 2], gz)
        return carry

    lax.fori_loop(0, _ROWS_PER_TILE, row_body, jnp.int32(0))

    pltpu.sync_copy(out3_v, out3_hbm.at[pl.ds(row0 * S * 3,
                                              _ROWS_PER_TILE * S * 3)])


def _sc_pass(d2_flat, thr_flat, cut_flat, px, py, pz, js):
    mesh = plsc.VectorSubcoreMesh(core_axis_name="c", subcore_axis_name="s")
    fn = pl.kernel(
        _sc_body,
        compiler_params=pltpu.CompilerParams(needs_layout_passes=False),
        out_type=[
            jax.ShapeDtypeStruct((B * P * S * 3,), jnp.float32),
            jax.ShapeDtypeStruct((B * P * K * 3,), jnp.float32),
        ],
        mesh=mesh,
        scratch_types=[
            pltpu.VMEM((N,), jnp.float32),
            pltpu.VMEM((N,), jnp.float32),
            pltpu.VMEM((N,), jnp.float32),
            pltpu.VMEM((_ROWS_PER_TILE, S), jnp.int32),
            pltpu.VMEM((_ROWS_PER_TILE + 16,), jnp.float32),
            pltpu.VMEM((_ROWS_PER_TILE + 16,), jnp.int32),
            pltpu.VMEM((N,), jnp.float32),
            pltpu.VMEM((K + 32,), jnp.float32),
            pltpu.VMEM((K + 32,), jnp.int32),
            pltpu.VMEM((K,), jnp.int32),
            pltpu.VMEM((_ROWS_PER_TILE * S * 3,), jnp.float32),
            pltpu.VMEM((K * 3,), jnp.float32),
        ],
    )
    return fn(d2_flat, thr_flat, cut_flat, px, py, pz, js)


_G = 32


def _tc2_body(g_ref, cov_ref):
    for i in range(_G):
        g = g_ref[i]
        m = jnp.mean(g, axis=0, keepdims=True)
        cg = g - m
        cov = lax.dot_general(cg, cg, (((0,), (0,)), ((), ())))
        cov_ref[i] = cov / float(K)


def _tc2_pass(g3):
    return pl.pallas_call(
        _tc2_body,
        grid=(B * P // _G,),
        in_specs=[pl.BlockSpec((_G, K, 3), lambda i: (i, 0, 0))],
        out_specs=pl.BlockSpec((_G, 3, 3), lambda i: (i, 0, 0)),
        out_shape=jax.ShapeDtypeStruct((B * P, 3, 3), jnp.float32),
    )(g3)


def kernel(pointclouds):
    cidx = jnp.linspace(0.0, N - 1, P).astype(jnp.int32)
    idx64 = jnp.linspace(0.0, float(P * K - 1), N).astype(jnp.int32)
    js = (idx64 % K).reshape(P, S)

    p3 = jnp.transpose(pointclouds, (0, 2, 1))
    c3 = pointclouds[:, cidx, :]

    d2all, thr, cut = _tc_pass(p3, c3)

    d2_flat = d2all.reshape(B * P, N)
    thr_flat = thr.reshape(B * P)
    cut_flat = cut.reshape(B * P)
    px = p3[:, 0, :]
    py = p3[:, 1, :]
    pz = p3[:, 2, :]

    out3_flat, g3_flat = _sc_pass(d2_flat, thr_flat, cut_flat, px, py, pz, js)

    out3 = out3_flat.reshape(B, P * S, 3)
    cov = _tc2_pass(g3_flat.reshape(B * P, K, 3)).reshape(B, P, 3, 3)

    _, frames = jnp.linalg.eigh(cov)
    main_axis = frames[:, :, :, 2]
    ax = jnp.repeat(main_axis, S, axis=1)
    return jnp.concatenate([out3, ax], axis=-1)
